# Initial kernel scaffold; baseline (speedup 1.0000x reference)
import functools

import jax
import jax.numpy as jnp
from jax import lax
from jax.experimental import pallas as pl
from jax.experimental.pallas import tpu as pltpu
from jax.experimental.pallas import tpu_sc as plsc

_N = 10000
_E = 320000
_HP = 32
_NC = 2
_NS = 16
_W = 2000
_RPT = _N // _NS
_EPT = _E // (_NC * _NS)


def _agg_body(tab_hbm, src_hbm, dst_hbm, zero_hbm, out_hbm,
              src_v, dst_v, rows_v, agg_sh, sem):
    cid = lax.axis_index("c")
    sid = lax.axis_index("s")
    pltpu.sync_copy(zero_hbm.at[pl.ds(sid * _RPT, _RPT)],
                    agg_sh.at[pl.ds(sid * _RPT, _RPT)])
    plsc.subcore_barrier()
    base = (cid * _NS + sid) * _EPT
    for w in range(_EPT // _W):
        off = base + w * _W
        pltpu.sync_copy(src_hbm.at[pl.ds(off, _W)], src_v)
        pltpu.sync_copy(dst_hbm.at[pl.ds(off, _W)], dst_v)
        pltpu.async_copy(tab_hbm.at[src_v], rows_v, sem).wait()
        pltpu.sync_copy(rows_v, agg_sh.at[dst_v], add=True)
    plsc.subcore_barrier()
    pltpu.sync_copy(agg_sh.at[pl.ds(sid * _RPT, _RPT)],
                    out_hbm.at[cid, pl.ds(sid * _RPT, _RPT)])


_agg = functools.partial(
    pl.kernel,
    out_type=jax.ShapeDtypeStruct((_NC, _N, _HP), jnp.float32),
    mesh=plsc.VectorSubcoreMesh(core_axis_name="c", subcore_axis_name="s"),
    scratch_types=[
        pltpu.VMEM((_W,), jnp.int32),
        pltpu.VMEM((_W,), jnp.int32),
        pltpu.VMEM((_W, _HP), jnp.float32),
        pltpu.VMEM_SHARED((_N, _HP), jnp.float32),
        pltpu.SemaphoreType.DMA,
    ],
)(_agg_body)


_BR = 1000


def _pre1_body(x_ref, w1t_ref, d0_ref, d1_ref, hp_ref, dinv_ref):
    deg = d0_ref[:, :1] + d1_ref[:, :1] + 1.0
    dinv = lax.rsqrt(deg)
    h = jnp.dot(x_ref[...], w1t_ref[...], preferred_element_type=jnp.float32)
    hp_ref[...] = h * dinv
    dinv_ref[...] = jnp.broadcast_to(dinv, (_BR, _HP))


def _pre1(x, w1t, deg0, deg1):
    return pl.pallas_call(
        _pre1_body,
        grid=(_N // _BR,),
        in_specs=[
            pl.BlockSpec((_BR, 128), lambda i: (i, 0)),
            pl.BlockSpec((128, _HP), lambda i: (0, 0)),
            pl.BlockSpec((_BR, _HP), lambda i: (i, 0)),
            pl.BlockSpec((_BR, _HP), lambda i: (i, 0)),
        ],
        out_specs=[pl.BlockSpec((_BR, _HP), lambda i: (i, 0))] * 2,
        out_shape=[jax.ShapeDtypeStruct((_N, _HP), jnp.float32)] * 2,
    )(x, w1t, deg0, deg1)


def _norm_relu(t):
    n2 = jnp.sum(t * t, axis=1, keepdims=True)
    inv = 1.0 / jnp.maximum(jnp.sqrt(n2), 1e-12)
    return jnp.maximum(t * inv, 0.0)


def _postpre_body(a0, a1, hp, dinv, b, res, wnt, out_ref, hpn_ref):
    t = (a0[...] + a1[...] + hp[...]) * dinv[...] + b[0:1, :]
    out = _norm_relu(t) + res[...]
    out_ref[...] = out
    hpn_ref[...] = (
        jnp.dot(out, wnt[...], preferred_element_type=jnp.float32) * dinv[...])


def _postpre(agg0, agg1, hp, dinvb, b, res, wnt):
    blk = pl.BlockSpec((_BR, _HP), lambda i: (i, 0))
    return pl.pallas_call(
        _postpre_body,
        grid=(_N // _BR,),
        in_specs=[blk, blk, blk, blk,
                  pl.BlockSpec((8, _HP), lambda i: (0, 0)),
                  blk,
                  pl.BlockSpec((_HP, _HP), lambda i: (0, 0))],
        out_specs=[blk, blk],
        out_shape=[jax.ShapeDtypeStruct((_N, _HP), jnp.float32)] * 2,
    )(agg0, agg1, hp, dinvb, b, res, wnt)


def _final_body(a0, a1, hp, dinv, b, res, wlt, bl, out_ref):
    t = (a0[...] + a1[...] + hp[...]) * dinv[...] + b[0:1, :]
    out6 = _norm_relu(t) + res[...]
    out_ref[...] = (
        jnp.dot(out6, wlt[...], preferred_element_type=jnp.float32)
        + bl[0:1, :])


def _final(agg0, agg1, hp, dinvb, b, res, wlt, bl):
    blk = pl.BlockSpec((_BR, _HP), lambda i: (i, 0))
    bspec = pl.BlockSpec((8, _HP), lambda i: (0, 0))
    return pl.pallas_call(
        _final_body,
        grid=(_N // _BR,),
        in_specs=[blk, blk, blk, blk, bspec, blk,
                  pl.BlockSpec((_HP, _HP), lambda i: (0, 0)), bspec],
        out_specs=blk,
        out_shape=jax.ShapeDtypeStruct((_N, _HP), jnp.float32),
    )(agg0, agg1, hp, dinvb, b, res, wlt, bl)


def _padw(Wm):
    o, i = Wm.shape
    ip = 128 if i == 128 else _HP
    return jnp.zeros((ip, _HP), jnp.float32).at[:i, :o].set(Wm.T)


def _padb(b):
    return jnp.zeros((8, _HP), jnp.float32).at[0, :b.shape[0]].set(b)


def kernel(x, edge_index, W1, b1, W2, b2, W3, b3, W4, b4, W5, b5,
           W6, b6, Wl, bl):
    src = edge_index[0]
    dst = edge_index[1]
    zeros32 = jnp.zeros((_N, _HP), jnp.float32)
    ones32 = jnp.ones((_N, _HP), jnp.float32)

    w2t, w3t, w4t, w5t, w6t = map(_padw, (W2, W3, W4, W5, W6))
    wlt = _padw(Wl)

    degt = _agg(ones32, src, dst, zeros32)
    hp1, dinvb = _pre1(x, _padw(W1), degt[0], degt[1])

    aggt = _agg(hp1, src, dst, zeros32)
    out1, hp2 = _postpre(aggt[0], aggt[1], hp1, dinvb, _padb(b1), zeros32, w2t)
    aggt = _agg(hp2, src, dst, zeros32)
    out2, hp3 = _postpre(aggt[0], aggt[1], hp2, dinvb, _padb(b2), out1, w3t)
    aggt = _agg(hp3, src, dst, zeros32)
    out3, hp4 = _postpre(aggt[0], aggt[1], hp3, dinvb, _padb(b3), out2, w4t)
    aggt = _agg(hp4, src, dst, zeros32)
    out4, hp5 = _postpre(aggt[0], aggt[1], hp4, dinvb, _padb(b4), out3, w5t)
    aggt = _agg(hp5, src, dst, zeros32)
    out5, hp6 = _postpre(aggt[0], aggt[1], hp5, dinvb, _padb(b5), out4, w6t)
    aggt = _agg(hp6, src, dst, zeros32)
    final = _final(aggt[0], aggt[1], hp6, dinvb, _padb(b6), out5, wlt,
                   _padb(bl))
    return final[:, :10]


# speedup vs baseline: 33.8770x; 33.8770x over previous
import functools

import jax
import jax.numpy as jnp
from jax import lax
from jax.experimental import pallas as pl
from jax.experimental.pallas import tpu as pltpu
from jax.experimental.pallas import tpu_sc as plsc

_N = 10000
_E = 320000
_HP = 32
_NC = 2
_NS = 16
_W = 2000
_RPT = 624
_TAIL = _N - _RPT * _NS
_EPT = _E // (_NC * _NS)


def _agg_body(tab_hbm, src_hbm, dst_hbm, zero_hbm, out_hbm,
              src_v, dst_v, rows_v, tab_sh, agg_sh, sem):
    cid = lax.axis_index("c")
    sid = lax.axis_index("s")
    pltpu.sync_copy(tab_hbm.at[pl.ds(sid * _RPT, _RPT)],
                    tab_sh.at[pl.ds(sid * _RPT, _RPT)])
    pltpu.sync_copy(zero_hbm.at[pl.ds(sid * _RPT, _RPT)],
                    agg_sh.at[pl.ds(sid * _RPT, _RPT)])

    @pl.when(sid == 0)
    def _():
        pltpu.sync_copy(tab_hbm.at[pl.ds(_RPT * _NS, _TAIL)],
                        tab_sh.at[pl.ds(_RPT * _NS, _TAIL)])
        pltpu.sync_copy(zero_hbm.at[pl.ds(_RPT * _NS, _TAIL)],
                        agg_sh.at[pl.ds(_RPT * _NS, _TAIL)])

    plsc.subcore_barrier()
    base = (cid * _NS + sid) * _EPT
    for w in range(_EPT // _W):
        off = base + w * _W
        pltpu.sync_copy(src_hbm.at[pl.ds(off, _W)], src_v)
        pltpu.sync_copy(dst_hbm.at[pl.ds(off, _W)], dst_v)
        pltpu.async_copy(tab_sh.at[src_v], rows_v, sem).wait()
        pltpu.sync_copy(rows_v, agg_sh.at[dst_v], add=True)
    plsc.subcore_barrier()
    pltpu.sync_copy(agg_sh.at[pl.ds(sid * _RPT, _RPT)],
                    out_hbm.at[cid, pl.ds(sid * _RPT, _RPT)])

    @pl.when(sid == 0)
    def _():
        pltpu.sync_copy(agg_sh.at[pl.ds(_RPT * _NS, _TAIL)],
                        out_hbm.at[cid, pl.ds(_RPT * _NS, _TAIL)])


@functools.cache
def _get_agg():
    return pl.kernel(
        _agg_body,
        out_type=jax.ShapeDtypeStruct((_NC, _N, _HP), jnp.float32),
        mesh=plsc.VectorSubcoreMesh(core_axis_name="c", subcore_axis_name="s",
                                    num_cores=_NC, num_subcores=_NS),
        scratch_types=[
            pltpu.VMEM((_W,), jnp.int32),
            pltpu.VMEM((_W,), jnp.int32),
            pltpu.VMEM((_W, _HP), jnp.float32),
            pltpu.VMEM_SHARED((_N, _HP), jnp.float32),
            pltpu.VMEM_SHARED((_N, _HP), jnp.float32),
            pltpu.SemaphoreType.DMA,
        ],
        compiler_params=pltpu.CompilerParams(use_tc_tiling_on_sc=False),
    )


def _agg(tab, src, dst, zeros):
    return _get_agg()(tab, src, dst, zeros)


_BR = 1000


def _pre1_body(x_ref, w1t_ref, d0_ref, d1_ref, hp_ref, dinv_ref):
    deg = d0_ref[:, :1] + d1_ref[:, :1] + 1.0
    dinv = lax.rsqrt(deg)
    h = jnp.dot(x_ref[...], w1t_ref[...], preferred_element_type=jnp.float32)
    hp_ref[...] = h * dinv
    dinv_ref[...] = jnp.broadcast_to(dinv, (_BR, _HP))


def _pre1(x, w1t, deg0, deg1):
    return pl.pallas_call(
        _pre1_body,
        grid=(_N // _BR,),
        in_specs=[
            pl.BlockSpec((_BR, 128), lambda i: (i, 0)),
            pl.BlockSpec((128, _HP), lambda i: (0, 0)),
            pl.BlockSpec((_BR, _HP), lambda i: (i, 0)),
            pl.BlockSpec((_BR, _HP), lambda i: (i, 0)),
        ],
        out_specs=[pl.BlockSpec((_BR, _HP), lambda i: (i, 0))] * 2,
        out_shape=[jax.ShapeDtypeStruct((_N, _HP), jnp.float32)] * 2,
    )(x, w1t, deg0, deg1)


def _norm_relu(t):
    n2 = jnp.sum(t * t, axis=1, keepdims=True)
    inv = 1.0 / jnp.maximum(jnp.sqrt(n2), 1e-12)
    return jnp.maximum(t * inv, 0.0)


def _postpre_body(a0, a1, hp, dinv, b, res, wnt, out_ref, hpn_ref):
    t = (a0[...] + a1[...] + hp[...]) * dinv[...] + b[0:1, :]
    out = _norm_relu(t) + res[...]
    out_ref[...] = out
    hpn_ref[...] = (
        jnp.dot(out, wnt[...], preferred_element_type=jnp.float32) * dinv[...])


def _postpre(agg0, agg1, hp, dinvb, b, res, wnt):
    blk = pl.BlockSpec((_BR, _HP), lambda i: (i, 0))
    return pl.pallas_call(
        _postpre_body,
        grid=(_N // _BR,),
        in_specs=[blk, blk, blk, blk,
                  pl.BlockSpec((8, _HP), lambda i: (0, 0)),
                  blk,
                  pl.BlockSpec((_HP, _HP), lambda i: (0, 0))],
        out_specs=[blk, blk],
        out_shape=[jax.ShapeDtypeStruct((_N, _HP), jnp.float32)] * 2,
    )(agg0, agg1, hp, dinvb, b, res, wnt)


def _final_body(a0, a1, hp, dinv, b, res, wlt, bl, out_ref):
    t = (a0[...] + a1[...] + hp[...]) * dinv[...] + b[0:1, :]
    out6 = _norm_relu(t) + res[...]
    out_ref[...] = (
        jnp.dot(out6, wlt[...], preferred_element_type=jnp.float32)
        + bl[0:1, :])


def _final(agg0, agg1, hp, dinvb, b, res, wlt, bl):
    blk = pl.BlockSpec((_BR, _HP), lambda i: (i, 0))
    bspec = pl.BlockSpec((8, _HP), lambda i: (0, 0))
    return pl.pallas_call(
        _final_body,
        grid=(_N // _BR,),
        in_specs=[blk, blk, blk, blk, bspec, blk,
                  pl.BlockSpec((_HP, _HP), lambda i: (0, 0)), bspec],
        out_specs=blk,
        out_shape=jax.ShapeDtypeStruct((_N, _HP), jnp.float32),
    )(agg0, agg1, hp, dinvb, b, res, wlt, bl)


def _padw(Wm):
    o, i = Wm.shape
    ip = 128 if i == 128 else _HP
    return jnp.zeros((ip, _HP), jnp.float32).at[:i, :o].set(Wm.T)


def _padb(b):
    return jnp.zeros((8, _HP), jnp.float32).at[0, :b.shape[0]].set(b)


def kernel(x, edge_index, W1, b1, W2, b2, W3, b3, W4, b4, W5, b5,
           W6, b6, Wl, bl):
    src = edge_index[0]
    dst = edge_index[1]
    zeros32 = jnp.zeros((_N, _HP), jnp.float32)
    ones32 = jnp.ones((_N, _HP), jnp.float32)

    w2t, w3t, w4t, w5t, w6t = map(_padw, (W2, W3, W4, W5, W6))
    wlt = _padw(Wl)

    degt = _agg(ones32, src, dst, zeros32)
    hp1, dinvb = _pre1(x, _padw(W1), degt[0], degt[1])

    aggt = _agg(hp1, src, dst, zeros32)
    out1, hp2 = _postpre(aggt[0], aggt[1], hp1, dinvb, _padb(b1), zeros32, w2t)
    aggt = _agg(hp2, src, dst, zeros32)
    out2, hp3 = _postpre(aggt[0], aggt[1], hp2, dinvb, _padb(b2), out1, w3t)
    aggt = _agg(hp3, src, dst, zeros32)
    out3, hp4 = _postpre(aggt[0], aggt[1], hp3, dinvb, _padb(b3), out2, w4t)
    aggt = _agg(hp4, src, dst, zeros32)
    out4, hp5 = _postpre(aggt[0], aggt[1], hp4, dinvb, _padb(b4), out3, w5t)
    aggt = _agg(hp5, src, dst, zeros32)
    out5, hp6 = _postpre(aggt[0], aggt[1], hp5, dinvb, _padb(b5), out4, w6t)
    aggt = _agg(hp6, src, dst, zeros32)
    final = _final(aggt[0], aggt[1], hp6, dinvb, _padb(b6), out5, wlt,
                   _padb(bl))
    return final[:, :10]


# speedup vs baseline: 38.7198x; 1.1430x over previous
import functools

import jax
import jax.numpy as jnp
from jax import lax
from jax.experimental import pallas as pl
from jax.experimental.pallas import tpu as pltpu
from jax.experimental.pallas import tpu_sc as plsc

_N = 10000
_E = 320000
_HP = 32
_NC = 2
_NS = 16
_W = 1000
_RPT = 624
_TAIL = _N - _RPT * _NS
_EPT = _E // (_NC * _NS)
_NWIN = _EPT // _W
_DW = 16


def _agg_body(tab_hbm, src_hbm, dst_hbm, zero_hbm, out_hbm,
              srcw_v, dstw_v, rows_v, tab_sh, agg_sh, gsem, ssem, stsem):
    cid = lax.axis_index("c")
    sid = lax.axis_index("s")
    tile = cid * _NS + sid
    stages = [
        pltpu.async_copy(tab_hbm.at[pl.ds(sid * _RPT, _RPT)],
                         tab_sh.at[pl.ds(sid * _RPT, _RPT)], stsem),
        pltpu.async_copy(zero_hbm.at[pl.ds(sid * _RPT, _RPT)],
                         agg_sh.at[pl.ds(sid * _RPT, _RPT)], stsem),
        pltpu.async_copy(src_hbm.at[tile], srcw_v, stsem),
        pltpu.async_copy(dst_hbm.at[tile], dstw_v, stsem),
    ]

    @pl.when(sid == 0)
    def _():
        pltpu.sync_copy(tab_hbm.at[pl.ds(_RPT * _NS, _TAIL)],
                        tab_sh.at[pl.ds(_RPT * _NS, _TAIL)])
        pltpu.sync_copy(zero_hbm.at[pl.ds(_RPT * _NS, _TAIL)],
                        agg_sh.at[pl.ds(_RPT * _NS, _TAIL)])

    for c in stages:
        c.wait()
    plsc.subcore_barrier()

    def gather(w):
        return pltpu.async_copy(tab_sh.at[srcw_v.at[w]],
                                rows_v.at[w % 2], gsem)

    def scatter(w):
        return pltpu.async_copy(rows_v.at[w % 2],
                                agg_sh.at[dstw_v.at[w]], ssem, add=True)

    g = gather(0)
    scats = {}
    for w in range(_NWIN):
        g.wait()
        scats[w] = scatter(w)
        if w + 1 < _NWIN:
            if w >= 1:
                scats.pop(w - 1).wait()
            g = gather(w + 1)
    for w in sorted(scats):
        scats.pop(w).wait()

    plsc.subcore_barrier()
    pltpu.sync_copy(agg_sh.at[pl.ds(sid * _RPT, _RPT)],
                    out_hbm.at[cid, pl.ds(sid * _RPT, _RPT)])

    @pl.when(sid == 0)
    def _():
        pltpu.sync_copy(agg_sh.at[pl.ds(_RPT * _NS, _TAIL)],
                        out_hbm.at[cid, pl.ds(_RPT * _NS, _TAIL)])


@functools.cache
def _get_agg():
    return pl.kernel(
        _agg_body,
        out_type=jax.ShapeDtypeStruct((_NC, _N, _HP), jnp.float32),
        mesh=plsc.VectorSubcoreMesh(core_axis_name="c", subcore_axis_name="s",
                                    num_cores=_NC, num_subcores=_NS),
        scratch_types=[
            pltpu.VMEM((_NWIN, _W), jnp.int32),
            pltpu.VMEM((_NWIN, _W), jnp.int32),
            pltpu.VMEM((2, _W, _HP), jnp.float32),
            pltpu.VMEM_SHARED((_N, _HP), jnp.float32),
            pltpu.VMEM_SHARED((_N, _HP), jnp.float32),
            pltpu.SemaphoreType.DMA,
            pltpu.SemaphoreType.DMA,
            pltpu.SemaphoreType.DMA,
        ],
        compiler_params=pltpu.CompilerParams(use_tc_tiling_on_sc=False),
    )


def _agg(tab, src3, dst3, zeros):
    return _get_agg()(tab, src3, dst3, zeros)


def _deg_body(dst_hbm, ones_hbm, zero_hbm, out_hbm,
              dstw_v, ones_v, deg_sh, ssem, stsem):
    cid = lax.axis_index("c")
    sid = lax.axis_index("s")
    tile = cid * _NS + sid
    stages = [
        pltpu.async_copy(zero_hbm.at[pl.ds(sid * _RPT, _RPT)],
                         deg_sh.at[pl.ds(sid * _RPT, _RPT)], stsem),
        pltpu.async_copy(dst_hbm.at[tile], dstw_v, stsem),
        pltpu.async_copy(ones_hbm, ones_v, stsem),
    ]

    @pl.when(sid == 0)
    def _():
        pltpu.sync_copy(zero_hbm.at[pl.ds(_RPT * _NS, _TAIL)],
                        deg_sh.at[pl.ds(_RPT * _NS, _TAIL)])

    for c in stages:
        c.wait()
    plsc.subcore_barrier()
    scats = [pltpu.async_copy(ones_v, deg_sh.at[dstw_v.at[w]], ssem,
                              add=True)
             for w in range(_NWIN)]
    for c in scats:
        c.wait()
    plsc.subcore_barrier()
    pltpu.sync_copy(deg_sh.at[pl.ds(sid * _RPT, _RPT)],
                    out_hbm.at[cid, pl.ds(sid * _RPT, _RPT)])

    @pl.when(sid == 0)
    def _():
        pltpu.sync_copy(deg_sh.at[pl.ds(_RPT * _NS, _TAIL)],
                        out_hbm.at[cid, pl.ds(_RPT * _NS, _TAIL)])


@functools.cache
def _get_deg():
    return pl.kernel(
        _deg_body,
        out_type=jax.ShapeDtypeStruct((_NC, _N, _DW), jnp.float32),
        mesh=plsc.VectorSubcoreMesh(core_axis_name="c", subcore_axis_name="s",
                                    num_cores=_NC, num_subcores=_NS),
        scratch_types=[
            pltpu.VMEM((_NWIN, _W), jnp.int32),
            pltpu.VMEM((_W, _DW), jnp.float32),
            pltpu.VMEM_SHARED((_N, _DW), jnp.float32),
            pltpu.SemaphoreType.DMA,
            pltpu.SemaphoreType.DMA,
        ],
        compiler_params=pltpu.CompilerParams(use_tc_tiling_on_sc=False),
    )


def _deg(dst3, ones, zeros16):
    return _get_deg()(dst3, ones, zeros16)


_BR = 1000


def _pre1_body(x_ref, w1t_ref, d0_ref, d1_ref, hp_ref, dinv_ref):
    deg = d0_ref[:, :1] + d1_ref[:, :1] + 1.0
    dinv = lax.rsqrt(deg)
    h = jnp.dot(x_ref[...], w1t_ref[...], preferred_element_type=jnp.float32)
    hp_ref[...] = h * dinv
    dinv_ref[...] = jnp.broadcast_to(dinv, (_BR, _HP))


def _pre1(x, w1t, deg0, deg1):
    return pl.pallas_call(
        _pre1_body,
        grid=(_N // _BR,),
        in_specs=[
            pl.BlockSpec((_BR, 128), lambda i: (i, 0)),
            pl.BlockSpec((128, _HP), lambda i: (0, 0)),
            pl.BlockSpec((_BR, _DW), lambda i: (i, 0)),
            pl.BlockSpec((_BR, _DW), lambda i: (i, 0)),
        ],
        out_specs=[pl.BlockSpec((_BR, _HP), lambda i: (i, 0))] * 2,
        out_shape=[jax.ShapeDtypeStruct((_N, _HP), jnp.float32)] * 2,
    )(x, w1t, deg0, deg1)


def _norm_relu(t):
    n2 = jnp.sum(t * t, axis=1, keepdims=True)
    inv = 1.0 / jnp.maximum(jnp.sqrt(n2), 1e-12)
    return jnp.maximum(t * inv, 0.0)


def _postpre_body(a0, a1, hp, dinv, b, res, wnt, out_ref, hpn_ref):
    t = (a0[...] + a1[...] + hp[...]) * dinv[...] + b[0:1, :]
    out = _norm_relu(t) + res[...]
    out_ref[...] = out
    hpn_ref[...] = (
        jnp.dot(out, wnt[...], preferred_element_type=jnp.float32) * dinv[...])


def _postpre(agg0, agg1, hp, dinvb, b, res, wnt):
    blk = pl.BlockSpec((_BR, _HP), lambda i: (i, 0))
    return pl.pallas_call(
        _postpre_body,
        grid=(_N // _BR,),
        in_specs=[blk, blk, blk, blk,
                  pl.BlockSpec((8, _HP), lambda i: (0, 0)),
                  blk,
                  pl.BlockSpec((_HP, _HP), lambda i: (0, 0))],
        out_specs=[blk, blk],
        out_shape=[jax.ShapeDtypeStruct((_N, _HP), jnp.float32)] * 2,
    )(agg0, agg1, hp, dinvb, b, res, wnt)


def _final_body(a0, a1, hp, dinv, b, res, wlt, bl, out_ref):
    t = (a0[...] + a1[...] + hp[...]) * dinv[...] + b[0:1, :]
    out6 = _norm_relu(t) + res[...]
    out_ref[...] = (
        jnp.dot(out6, wlt[...], preferred_element_type=jnp.float32)
        + bl[0:1, :])


def _final(agg0, agg1, hp, dinvb, b, res, wlt, bl):
    blk = pl.BlockSpec((_BR, _HP), lambda i: (i, 0))
    bspec = pl.BlockSpec((8, _HP), lambda i: (0, 0))
    return pl.pallas_call(
        _final_body,
        grid=(_N // _BR,),
        in_specs=[blk, blk, blk, blk, bspec, blk,
                  pl.BlockSpec((_HP, _HP), lambda i: (0, 0)), bspec],
        out_specs=blk,
        out_shape=jax.ShapeDtypeStruct((_N, _HP), jnp.float32),
    )(agg0, agg1, hp, dinvb, b, res, wlt, bl)


def _padw(Wm):
    o, i = Wm.shape
    ip = 128 if i == 128 else _HP
    return jnp.zeros((ip, _HP), jnp.float32).at[:i, :o].set(Wm.T)


def _padb(b):
    return jnp.zeros((8, _HP), jnp.float32).at[0, :b.shape[0]].set(b)


def kernel(x, edge_index, W1, b1, W2, b2, W3, b3, W4, b4, W5, b5,
           W6, b6, Wl, bl):
    src = edge_index[0].reshape(_NC * _NS, _NWIN, _W)
    dst = edge_index[1].reshape(_NC * _NS, _NWIN, _W)
    zeros32 = jnp.zeros((_N, _HP), jnp.float32)
    zeros16 = jnp.zeros((_N, _DW), jnp.float32)
    ones = jnp.ones((_W, _DW), jnp.float32)

    w2t, w3t, w4t, w5t, w6t = map(_padw, (W2, W3, W4, W5, W6))
    wlt = _padw(Wl)

    degt = _deg(dst, ones, zeros16)
    hp1, dinvb = _pre1(x, _padw(W1), degt[0], degt[1])

    aggt = _agg(hp1, src, dst, zeros32)
    out1, hp2 = _postpre(aggt[0], aggt[1], hp1, dinvb, _padb(b1), zeros32, w2t)
    aggt = _agg(hp2, src, dst, zeros32)
    out2, hp3 = _postpre(aggt[0], aggt[1], hp2, dinvb, _padb(b2), out1, w3t)
    aggt = _agg(hp3, src, dst, zeros32)
    out3, hp4 = _postpre(aggt[0], aggt[1], hp3, dinvb, _padb(b3), out2, w4t)
    aggt = _agg(hp4, src, dst, zeros32)
    out4, hp5 = _postpre(aggt[0], aggt[1], hp4, dinvb, _padb(b4), out3, w5t)
    aggt = _agg(hp5, src, dst, zeros32)
    out5, hp6 = _postpre(aggt[0], aggt[1], hp5, dinvb, _padb(b5), out4, w6t)
    aggt = _agg(hp6, src, dst, zeros32)
    final = _final(aggt[0], aggt[1], hp6, dinvb, _padb(b6), out5, wlt,
                   _padb(bl))
    return final[:, :10]


# speedup vs baseline: 61.8788x; 1.5981x over previous
import functools

import jax
import jax.numpy as jnp
from jax import lax
from jax.experimental import pallas as pl
from jax.experimental.pallas import tpu as pltpu
from jax.experimental.pallas import tpu_sc as plsc

_N = 10000
_E = 320000
_HP = 32
_NC = 2
_NS = 16
_W = 1000
_RPT = 624
_TAIL = _N - _RPT * _NS
_EPT = _E // (_NC * _NS)
_NWIN = _EPT // _W
_DW = 32


def _agg_body(tab_hbm, src_hbm, dst_hbm, zero_hbm, out0_hbm, out1_hbm,
              srcw_v, dstw_v, rows_v, tab_sh, agg_sh, gsem, ssem, stsem):
    cid = lax.axis_index("c")
    sid = lax.axis_index("s")
    tile = cid * _NS + sid
    stages = [
        pltpu.async_copy(tab_hbm.at[pl.ds(sid * _RPT, _RPT)],
                         tab_sh.at[pl.ds(sid * _RPT, _RPT)], stsem),
        pltpu.async_copy(zero_hbm.at[pl.ds(sid * _RPT, _RPT)],
                         agg_sh.at[pl.ds(sid * _RPT, _RPT)], stsem),
        pltpu.async_copy(src_hbm.at[tile], srcw_v, stsem),
        pltpu.async_copy(dst_hbm.at[tile], dstw_v, stsem),
    ]

    @pl.when(sid == 0)
    def _():
        pltpu.sync_copy(tab_hbm.at[pl.ds(_RPT * _NS, _TAIL)],
                        tab_sh.at[pl.ds(_RPT * _NS, _TAIL)])
        pltpu.sync_copy(zero_hbm.at[pl.ds(_RPT * _NS, _TAIL)],
                        agg_sh.at[pl.ds(_RPT * _NS, _TAIL)])

    for c in stages:
        c.wait()
    plsc.subcore_barrier()

    def gather(w):
        return pltpu.async_copy(tab_sh.at[srcw_v.at[w]],
                                rows_v.at[w % 2], gsem)

    def scatter(w):
        return pltpu.async_copy(rows_v.at[w % 2],
                                agg_sh.at[dstw_v.at[w]], ssem, add=True)

    g = gather(0)
    scats = {}
    for w in range(_NWIN):
        g.wait()
        scats[w] = scatter(w)
        if w + 1 < _NWIN:
            if w >= 1:
                scats.pop(w - 1).wait()
            g = gather(w + 1)
    for w in sorted(scats):
        scats.pop(w).wait()

    plsc.subcore_barrier()

    @pl.when(cid == 0)
    def _():
        pltpu.sync_copy(agg_sh.at[pl.ds(sid * _RPT, _RPT)],
                        out0_hbm.at[pl.ds(sid * _RPT, _RPT)])

    @pl.when(cid == 1)
    def _():
        pltpu.sync_copy(agg_sh.at[pl.ds(sid * _RPT, _RPT)],
                        out1_hbm.at[pl.ds(sid * _RPT, _RPT)])

    @pl.when((sid == 0) & (cid == 0))
    def _():
        pltpu.sync_copy(agg_sh.at[pl.ds(_RPT * _NS, _TAIL)],
                        out0_hbm.at[pl.ds(_RPT * _NS, _TAIL)])

    @pl.when((sid == 0) & (cid == 1))
    def _():
        pltpu.sync_copy(agg_sh.at[pl.ds(_RPT * _NS, _TAIL)],
                        out1_hbm.at[pl.ds(_RPT * _NS, _TAIL)])


@functools.cache
def _get_agg():
    return pl.kernel(
        _agg_body,
        out_type=[jax.ShapeDtypeStruct((_N, _HP), jnp.float32)] * 2,
        mesh=plsc.VectorSubcoreMesh(core_axis_name="c", subcore_axis_name="s",
                                    num_cores=_NC, num_subcores=_NS),
        scratch_types=[
            pltpu.VMEM((_NWIN, _W), jnp.int32),
            pltpu.VMEM((_NWIN, _W), jnp.int32),
            pltpu.VMEM((2, _W, _HP), jnp.float32),
            pltpu.VMEM_SHARED((_N, _HP), jnp.float32),
            pltpu.VMEM_SHARED((_N, _HP), jnp.float32),
            pltpu.SemaphoreType.DMA,
            pltpu.SemaphoreType.DMA,
            pltpu.SemaphoreType.DMA,
        ],
        compiler_params=pltpu.CompilerParams(use_tc_tiling_on_sc=False),
    )


def _agg(tab, src3, dst3, zeros):
    return _get_agg()(tab, src3, dst3, zeros)


def _deg_body(dst_hbm, ones_hbm, zero_hbm, out0_hbm, out1_hbm,
              dstw_v, ones_v, deg_sh, ssem, stsem):
    cid = lax.axis_index("c")
    sid = lax.axis_index("s")
    tile = cid * _NS + sid
    stages = [
        pltpu.async_copy(zero_hbm.at[pl.ds(sid * _RPT, _RPT)],
                         deg_sh.at[pl.ds(sid * _RPT, _RPT)], stsem),
        pltpu.async_copy(dst_hbm.at[tile], dstw_v, stsem),
        pltpu.async_copy(ones_hbm, ones_v, stsem),
    ]

    @pl.when(sid == 0)
    def _():
        pltpu.sync_copy(zero_hbm.at[pl.ds(_RPT * _NS, _TAIL)],
                        deg_sh.at[pl.ds(_RPT * _NS, _TAIL)])

    for c in stages:
        c.wait()
    plsc.subcore_barrier()
    scats = [pltpu.async_copy(ones_v, deg_sh.at[dstw_v.at[w]], ssem,
                              add=True)
             for w in range(_NWIN)]
    for c in scats:
        c.wait()
    plsc.subcore_barrier()

    @pl.when(cid == 0)
    def _():
        pltpu.sync_copy(deg_sh.at[pl.ds(sid * _RPT, _RPT)],
                        out0_hbm.at[pl.ds(sid * _RPT, _RPT)])

    @pl.when(cid == 1)
    def _():
        pltpu.sync_copy(deg_sh.at[pl.ds(sid * _RPT, _RPT)],
                        out1_hbm.at[pl.ds(sid * _RPT, _RPT)])

    @pl.when((sid == 0) & (cid == 0))
    def _():
        pltpu.sync_copy(deg_sh.at[pl.ds(_RPT * _NS, _TAIL)],
                        out0_hbm.at[pl.ds(_RPT * _NS, _TAIL)])

    @pl.when((sid == 0) & (cid == 1))
    def _():
        pltpu.sync_copy(deg_sh.at[pl.ds(_RPT * _NS, _TAIL)],
                        out1_hbm.at[pl.ds(_RPT * _NS, _TAIL)])


@functools.cache
def _get_deg():
    return pl.kernel(
        _deg_body,
        out_type=[jax.ShapeDtypeStruct((_N, _DW), jnp.float32)] * 2,
        mesh=plsc.VectorSubcoreMesh(core_axis_name="c", subcore_axis_name="s",
                                    num_cores=_NC, num_subcores=_NS),
        scratch_types=[
            pltpu.VMEM((_NWIN, _W), jnp.int32),
            pltpu.VMEM((_W, _DW), jnp.float32),
            pltpu.VMEM_SHARED((_N, _DW), jnp.float32),
            pltpu.SemaphoreType.DMA,
            pltpu.SemaphoreType.DMA,
        ],
        compiler_params=pltpu.CompilerParams(use_tc_tiling_on_sc=False),
    )


def _deg(dst3, ones, zeros16):
    return _get_deg()(dst3, ones, zeros16)


_NP = _N * _HP // 128
_BRP = _NP


def _pre1_body(x4_ref, w1bd_ref, d0_ref, d1_ref, hp_ref, dinv_ref):
    dinv = lax.rsqrt(d0_ref[...] + d1_ref[...] + 1.0)
    h = jnp.dot(x4_ref[...], w1bd_ref[...],
                preferred_element_type=jnp.float32)
    hp_ref[...] = h * dinv
    dinv_ref[...] = dinv


def _pre1(x4, w1bd, d0p, d1p):
    blk = pl.BlockSpec((_BRP, 128), lambda i: (i, 0))
    return pl.pallas_call(
        _pre1_body,
        grid=(1,),
        in_specs=[
            pl.BlockSpec((_BRP, 512), lambda i: (i, 0)),
            pl.BlockSpec((512, 128), lambda i: (0, 0)),
            blk, blk,
        ],
        out_specs=[blk, blk],
        out_shape=[jax.ShapeDtypeStruct((_NP, 128), jnp.float32)] * 2,
    )(x4, w1bd, d0p, d1p)


def _norm_relu(t, mseg):
    n2 = jnp.dot(t * t, mseg, preferred_element_type=jnp.float32)
    inv = 1.0 / jnp.maximum(jnp.sqrt(n2), 1e-12)
    return jnp.maximum(t * inv, 0.0)


def _postpre_body(a0, a1, hp, dinv, b, res, wbd, mseg, out_ref, hpn_ref):
    t = (a0[...] + a1[...] + hp[...]) * dinv[...] + b[0:1, :]
    out = _norm_relu(t, mseg[...]) + res[...]
    out_ref[...] = out
    hpn_ref[...] = (
        jnp.dot(out, wbd[...], preferred_element_type=jnp.float32)
        * dinv[...])


def _postpre(a0p, a1p, hpp, dinvp, b, resp, wbd, mseg):
    blk = pl.BlockSpec((_BRP, 128), lambda i: (i, 0))
    cst = pl.BlockSpec((128, 128), lambda i: (0, 0))
    return pl.pallas_call(
        _postpre_body,
        grid=(1,),
        in_specs=[blk, blk, blk, blk,
                  pl.BlockSpec((8, 128), lambda i: (0, 0)),
                  blk, cst, cst],
        out_specs=[blk, blk],
        out_shape=[jax.ShapeDtypeStruct((_NP, 128), jnp.float32)] * 2,
    )(a0p, a1p, hpp, dinvp, b, resp, wbd, mseg)


def _final_body(a0, a1, hp, dinv, b, res, wlbd, bl, mseg, out_ref):
    t = (a0[...] + a1[...] + hp[...]) * dinv[...] + b[0:1, :]
    out6 = _norm_relu(t, mseg[...]) + res[...]
    out_ref[...] = (
        jnp.dot(out6, wlbd[...], preferred_element_type=jnp.float32)
        + bl[0:1, :])


def _final(a0p, a1p, hpp, dinvp, b, resp, wlbd, bl, mseg):
    blk = pl.BlockSpec((_BRP, 128), lambda i: (i, 0))
    cst = pl.BlockSpec((128, 128), lambda i: (0, 0))
    bspec = pl.BlockSpec((8, 128), lambda i: (0, 0))
    return pl.pallas_call(
        _final_body,
        grid=(1,),
        in_specs=[blk, blk, blk, blk, bspec, blk, cst, bspec, cst],
        out_specs=blk,
        out_shape=jax.ShapeDtypeStruct((_NP, 128), jnp.float32),
    )(a0p, a1p, hpp, dinvp, b, resp, wlbd, bl, mseg)


def _padw_bd(Wm):
    o, i = Wm.shape
    ip = 128 if i == 128 else _HP
    wt = jnp.zeros((ip, _HP), jnp.float32).at[:i, :o].set(Wm.T)
    return jnp.kron(jnp.eye(4, dtype=jnp.float32), wt)


def _padb_p(b):
    bp = jnp.zeros((_HP,), jnp.float32).at[:b.shape[0]].set(b)
    return jnp.zeros((8, 128), jnp.float32).at[0, :].set(jnp.tile(bp, 4))


def kernel(x, edge_index, W1, b1, W2, b2, W3, b3, W4, b4, W5, b5,
           W6, b6, Wl, bl):
    src = edge_index[0].reshape(_NC * _NS, _NWIN, _W)
    dst = edge_index[1].reshape(_NC * _NS, _NWIN, _W)
    zeros32 = jnp.zeros((_N, _HP), jnp.float32)
    zerosp = jnp.zeros((_NP, 128), jnp.float32)
    ones_deg = jnp.ones((_W, _DW), jnp.float32)
    mseg = jnp.kron(jnp.eye(4, dtype=jnp.float32),
                    jnp.ones((_HP, _HP), jnp.float32))

    w2bd, w3bd, w4bd, w5bd, w6bd = map(_padw_bd, (W2, W3, W4, W5, W6))
    wlbd = _padw_bd(Wl)

    def packed(a):
        return a.reshape(_NP, 128)

    d0, d1 = _deg(dst, ones_deg, zeros32)
    hp1p, dinvp = _pre1(x.reshape(_NP, 512), _padw_bd(W1),
                        packed(d0), packed(d1))

    def unpacked(ap):
        return ap.reshape(_N, _HP)

    a0, a1 = _agg(unpacked(hp1p), src, dst, zeros32)
    out1p, hp2p = _postpre(packed(a0), packed(a1), hp1p, dinvp,
                           _padb_p(b1), zerosp, w2bd, mseg)
    a0, a1 = _agg(unpacked(hp2p), src, dst, zeros32)
    out2p, hp3p = _postpre(packed(a0), packed(a1), hp2p, dinvp,
                           _padb_p(b2), out1p, w3bd, mseg)
    a0, a1 = _agg(unpacked(hp3p), src, dst, zeros32)
    out3p, hp4p = _postpre(packed(a0), packed(a1), hp3p, dinvp,
                           _padb_p(b3), out2p, w4bd, mseg)
    a0, a1 = _agg(unpacked(hp4p), src, dst, zeros32)
    out4p, hp5p = _postpre(packed(a0), packed(a1), hp4p, dinvp,
                           _padb_p(b4), out3p, w5bd, mseg)
    a0, a1 = _agg(unpacked(hp5p), src, dst, zeros32)
    out5p, hp6p = _postpre(packed(a0), packed(a1), hp5p, dinvp,
                           _padb_p(b5), out4p, w6bd, mseg)
    a0, a1 = _agg(unpacked(hp6p), src, dst, zeros32)
    finalp = _final(packed(a0), packed(a1), hp6p, dinvp,
                    _padb_p(b6), out5p, wlbd, _padb_p(bl), mseg)
    return finalp.reshape(_N, _HP)[:, :10]
